# baseline (speedup 1.0000x reference)
import functools

import jax
import jax.numpy as jnp
from jax import lax
from jax.experimental import pallas as pl
from jax.experimental.pallas import tpu as pltpu
from jax.experimental.pallas import tpu_sc as plsc

L = 16
NW = 32
Z, X0, X1, X2 = 64, 128, 128, 128
P1, P2, P3 = 64, 128, 256
N = 50000
NBLK = N // L
BPW = -(-NBLK // NW)
NPAD = NW * BPW * L


def _sc_body(x0_hbm, i0_hbm, x1_hbm,
             t1w_h, t1j1_h, t1o_h, t1c_h,
             t2w_h, t2j1_h, t2j2_h, t2o_h, t2c_h,
             t3w_h, t3j1_h, t3j2_h, t3j3_h, t3o_h, t3c_h,
             out_hbm,
             x0_v, i0_v, x1_v, out_v,
             t1w, t1j1, t1o, t1c,
             t2w, t2j1, t2j2, t2o, t2c,
             t3w, t3j1, t3j2, t3j3, t3o, t3c):
    c = lax.axis_index("c")
    s = lax.axis_index("s")
    wid = s * 2 + c

    pltpu.sync_copy(x0_hbm, x0_v)
    pltpu.sync_copy(t1w_h, t1w)
    pltpu.sync_copy(t1j1_h, t1j1)
    pltpu.sync_copy(t1o_h, t1o)
    pltpu.sync_copy(t1c_h, t1c)
    pltpu.sync_copy(t2w_h, t2w)
    pltpu.sync_copy(t2j1_h, t2j1)
    pltpu.sync_copy(t2j2_h, t2j2)
    pltpu.sync_copy(t2o_h, t2o)
    pltpu.sync_copy(t2c_h, t2c)
    pltpu.sync_copy(t3w_h, t3w)
    pltpu.sync_copy(t3j1_h, t3j1)
    pltpu.sync_copy(t3j2_h, t3j2)
    pltpu.sync_copy(t3j3_h, t3j3)
    pltpu.sync_copy(t3o_h, t3o)
    pltpu.sync_copy(t3c_h, t3c)

    start = wid * BPW
    cnt = jnp.minimum(BPW, NBLK - start)
    pltpu.sync_copy(i0_hbm.at[pl.ds(start * L, BPW * L)], i0_v)

    zero16 = jnp.zeros((L,), jnp.float32)

    def blk_body(t, _):
        blk = start + t
        pltpu.sync_copy(x1_hbm.at[pl.ds(blk * (L * X1), L * X1)], x1_v)
        i0x = i0_v[pl.ds(t * L, L)] * X0

        def zbody(k, _):
            out_v[pl.ds(k * L, L)] = zero16
            return 0
        lax.fori_loop(0, (L * X2) // L, zbody, 0)

        def p1(p, _):
            b = p * L
            a = plsc.load_gather(x0_v, [i0x + t1w[pl.ds(b, L)]])
            v = (t1c[pl.ds(b, L)] * a) * plsc.load_gather(x1_v, [t1j1[pl.ds(b, L)]])
            plsc.addupdate_scatter(out_v, [t1o[pl.ds(b, L)]], v)
            return 0
        lax.fori_loop(0, P1, p1, 0)

        def p2(p, _):
            b = p * L
            a = plsc.load_gather(x0_v, [i0x + t2w[pl.ds(b, L)]])
            v = (t2c[pl.ds(b, L)] * a) * plsc.load_gather(x1_v, [t2j1[pl.ds(b, L)]])
            v = v * plsc.load_gather(x1_v, [t2j2[pl.ds(b, L)]])
            plsc.addupdate_scatter(out_v, [t2o[pl.ds(b, L)]], v)
            return 0
        lax.fori_loop(0, P2, p2, 0)

        def p3(p, _):
            b = p * L
            a = plsc.load_gather(x0_v, [i0x + t3w[pl.ds(b, L)]])
            v = (t3c[pl.ds(b, L)] * a) * plsc.load_gather(x1_v, [t3j1[pl.ds(b, L)]])
            v = v * (plsc.load_gather(x1_v, [t3j2[pl.ds(b, L)]]) *
                     plsc.load_gather(x1_v, [t3j3[pl.ds(b, L)]]))
            plsc.addupdate_scatter(out_v, [t3o[pl.ds(b, L)]], v)
            return 0
        lax.fori_loop(0, P3, p3, 0)

        pltpu.sync_copy(out_v, out_hbm.at[pl.ds(blk * (L * X2), L * X2)])
        return 0

    lax.fori_loop(0, cnt, blk_body, 0)


@jax.jit
def kernel(x0, i0, x1, idx1, coeff1, idx2, coeff2, idx3, coeff3):
    n = x1.shape[0]
    f32, i32 = jnp.float32, jnp.int32
    lane = jnp.arange(L, dtype=i32)

    def flat(j):
        return (lane[None, :] * X1 + j[:, None].astype(i32)).reshape(-1)

    def splat(v):
        return jnp.broadcast_to(v[:, None], (v.shape[0], L)).reshape(-1)

    args = [
        x0.reshape(-1), jnp.pad(i0, (0, NPAD - n)), x1.reshape(-1),
        splat(idx1[:, 0]), flat(idx1[:, 1]), flat(idx1[:, 2]), splat(coeff1),
        splat(idx2[:, 0]), flat(idx2[:, 1]), flat(idx2[:, 2]), flat(idx2[:, 3]),
        splat(coeff2),
        splat(idx3[:, 0]), flat(idx3[:, 1]), flat(idx3[:, 2]), flat(idx3[:, 3]),
        flat(idx3[:, 4]), splat(coeff3),
    ]

    run = pl.kernel(
        _sc_body,
        out_type=jax.ShapeDtypeStruct((n * X2,), f32),
        mesh=plsc.VectorSubcoreMesh(core_axis_name="c", subcore_axis_name="s"),
        compiler_params=pltpu.CompilerParams(needs_layout_passes=False),
        scratch_types=[
            pltpu.VMEM((Z * X0,), f32),
            pltpu.VMEM((BPW * L,), i32),
            pltpu.VMEM((L * X1,), f32),
            pltpu.VMEM((L * X2,), f32),
            pltpu.VMEM((P1 * L,), i32), pltpu.VMEM((P1 * L,), i32),
            pltpu.VMEM((P1 * L,), i32), pltpu.VMEM((P1 * L,), f32),
            pltpu.VMEM((P2 * L,), i32), pltpu.VMEM((P2 * L,), i32),
            pltpu.VMEM((P2 * L,), i32), pltpu.VMEM((P2 * L,), i32),
            pltpu.VMEM((P2 * L,), f32),
            pltpu.VMEM((P3 * L,), i32), pltpu.VMEM((P3 * L,), i32),
            pltpu.VMEM((P3 * L,), i32), pltpu.VMEM((P3 * L,), i32),
            pltpu.VMEM((P3 * L,), i32), pltpu.VMEM((P3 * L,), f32),
        ],
    )
    return run(*args).reshape(n, X2)


# speedup vs baseline: 1.3265x; 1.3265x over previous
import functools

import jax
import jax.numpy as jnp
from jax import lax
from jax.experimental import pallas as pl
from jax.experimental.pallas import tpu as pltpu
from jax.experimental.pallas import tpu_sc as plsc

L = 16
NW = 32
Z, X0, X1, X2 = 64, 128, 128, 128
P1, P2, P3 = 64, 128, 256
N = 50000
NBLK = N // L
BPW = -(-NBLK // NW)
NPAD = NW * BPW * L


def _sc_body(x0_hbm, i0_hbm, x1_hbm,
             t1w_h, t1j1_h, t1o_h, t1c_h,
             t2w_h, t2j1_h, t2j2_h, t2o_h, t2c_h,
             t3w_h, t3j1_h, t3j2_h, t3j3_h, t3o_h, t3c_h,
             out_hbm,
             x0_v, i0_v, x1_v, out_v,
             t1w, t1j1, t1o, t1c,
             t2w, t2j1, t2j2, t2o, t2c,
             t3w, t3j1, t3j2, t3j3, t3o, t3c):
    c = lax.axis_index("c")
    s = lax.axis_index("s")
    wid = s * 2 + c

    pltpu.sync_copy(x0_hbm, x0_v)
    pltpu.sync_copy(t1w_h, t1w)
    pltpu.sync_copy(t1j1_h, t1j1)
    pltpu.sync_copy(t1o_h, t1o)
    pltpu.sync_copy(t1c_h, t1c)
    pltpu.sync_copy(t2w_h, t2w)
    pltpu.sync_copy(t2j1_h, t2j1)
    pltpu.sync_copy(t2j2_h, t2j2)
    pltpu.sync_copy(t2o_h, t2o)
    pltpu.sync_copy(t2c_h, t2c)
    pltpu.sync_copy(t3w_h, t3w)
    pltpu.sync_copy(t3j1_h, t3j1)
    pltpu.sync_copy(t3j2_h, t3j2)
    pltpu.sync_copy(t3j3_h, t3j3)
    pltpu.sync_copy(t3o_h, t3o)
    pltpu.sync_copy(t3c_h, t3c)

    start = wid * BPW
    cnt = jnp.minimum(BPW, NBLK - start)
    pltpu.sync_copy(i0_hbm.at[pl.ds(start * L, BPW * L)], i0_v)

    zero16 = jnp.zeros((L,), jnp.float32)

    def blk_body(t, _):
        blk = start + t
        pltpu.sync_copy(x1_hbm.at[pl.ds(blk * (L * X1), L * X1)], x1_v)
        i0x = i0_v[pl.ds(t * L, L)] * X0

        @plsc.parallel_loop(0, L * X2, L, unroll=8)
        def _z(k):
            out_v[pl.ds(k, L)] = zero16

        @plsc.parallel_loop(0, P1 * L, L, unroll=4)
        def _p1(b):
            a = plsc.load_gather(x0_v, [i0x + t1w[pl.ds(b, L)]])
            v = (t1c[pl.ds(b, L)] * a) * plsc.load_gather(x1_v, [t1j1[pl.ds(b, L)]])
            plsc.addupdate_scatter(out_v, [t1o[pl.ds(b, L)]], v)

        @plsc.parallel_loop(0, P2 * L, L, unroll=4)
        def _p2(b):
            a = plsc.load_gather(x0_v, [i0x + t2w[pl.ds(b, L)]])
            v = (t2c[pl.ds(b, L)] * a) * plsc.load_gather(x1_v, [t2j1[pl.ds(b, L)]])
            v = v * plsc.load_gather(x1_v, [t2j2[pl.ds(b, L)]])
            plsc.addupdate_scatter(out_v, [t2o[pl.ds(b, L)]], v)

        @plsc.parallel_loop(0, P3 * L, L, unroll=4)
        def _p3(b):
            a = plsc.load_gather(x0_v, [i0x + t3w[pl.ds(b, L)]])
            v = (t3c[pl.ds(b, L)] * a) * plsc.load_gather(x1_v, [t3j1[pl.ds(b, L)]])
            v = v * (plsc.load_gather(x1_v, [t3j2[pl.ds(b, L)]]) *
                     plsc.load_gather(x1_v, [t3j3[pl.ds(b, L)]]))
            plsc.addupdate_scatter(out_v, [t3o[pl.ds(b, L)]], v)

        pltpu.sync_copy(out_v, out_hbm.at[pl.ds(blk * (L * X2), L * X2)])
        return 0

    lax.fori_loop(0, cnt, blk_body, 0)


@jax.jit
def kernel(x0, i0, x1, idx1, coeff1, idx2, coeff2, idx3, coeff3):
    n = x1.shape[0]
    f32, i32 = jnp.float32, jnp.int32
    lane = jnp.arange(L, dtype=i32)

    def flat(j):
        return (lane[None, :] * X1 + j[:, None].astype(i32)).reshape(-1)

    def splat(v):
        return jnp.broadcast_to(v[:, None], (v.shape[0], L)).reshape(-1)

    args = [
        x0.reshape(-1), jnp.pad(i0, (0, NPAD - n)), x1.reshape(-1),
        splat(idx1[:, 0]), flat(idx1[:, 1]), flat(idx1[:, 2]), splat(coeff1),
        splat(idx2[:, 0]), flat(idx2[:, 1]), flat(idx2[:, 2]), flat(idx2[:, 3]),
        splat(coeff2),
        splat(idx3[:, 0]), flat(idx3[:, 1]), flat(idx3[:, 2]), flat(idx3[:, 3]),
        flat(idx3[:, 4]), splat(coeff3),
    ]

    run = pl.kernel(
        _sc_body,
        out_type=jax.ShapeDtypeStruct((n * X2,), f32),
        mesh=plsc.VectorSubcoreMesh(core_axis_name="c", subcore_axis_name="s"),
        compiler_params=pltpu.CompilerParams(needs_layout_passes=False),
        scratch_types=[
            pltpu.VMEM((Z * X0,), f32),
            pltpu.VMEM((BPW * L,), i32),
            pltpu.VMEM((L * X1,), f32),
            pltpu.VMEM((L * X2,), f32),
            pltpu.VMEM((P1 * L,), i32), pltpu.VMEM((P1 * L,), i32),
            pltpu.VMEM((P1 * L,), i32), pltpu.VMEM((P1 * L,), f32),
            pltpu.VMEM((P2 * L,), i32), pltpu.VMEM((P2 * L,), i32),
            pltpu.VMEM((P2 * L,), i32), pltpu.VMEM((P2 * L,), i32),
            pltpu.VMEM((P2 * L,), f32),
            pltpu.VMEM((P3 * L,), i32), pltpu.VMEM((P3 * L,), i32),
            pltpu.VMEM((P3 * L,), i32), pltpu.VMEM((P3 * L,), i32),
            pltpu.VMEM((P3 * L,), i32), pltpu.VMEM((P3 * L,), f32),
        ],
    )
    return run(*args).reshape(n, X2)


# speedup vs baseline: 1.4490x; 1.0924x over previous
import functools

import jax
import jax.numpy as jnp
from jax import lax
from jax.experimental import pallas as pl
from jax.experimental.pallas import tpu as pltpu
from jax.experimental.pallas import tpu_sc as plsc

L = 16
NW = 32
Z, X0, X1, X2 = 64, 128, 128, 128
P1, P2, P3 = 64, 128, 256
N = 50000
NBLK = N // L
BPW = -(-NBLK // NW)
R = 7
NSB = BPW // R
NPAD = NW * BPW * L
BW = L * X1


def _sc_body(x0_hbm, i0_hbm, x1_hbm,
             t1w_h, t1j1_h, t1o_h, t1c_h,
             t2w_h, t2j1_h, t2j2_h, t2o_h, t2c_h,
             t3w_h, t3j1_h, t3j2_h, t3j3_h, t3o_h, t3c_h,
             out_hbm,
             x0_v, i0_v, x1_v, out_v,
             t1w, t1j1, t1o, t1c,
             t2w, t2j1, t2j2, t2o, t2c,
             t3w, t3j1, t3j2, t3j3, t3o, t3c):
    c = lax.axis_index("c")
    s = lax.axis_index("s")
    wid = s * 2 + c

    pltpu.sync_copy(x0_hbm, x0_v)
    for src, dst in ((t1w_h, t1w), (t1j1_h, t1j1), (t1o_h, t1o), (t1c_h, t1c),
                     (t2w_h, t2w), (t2j1_h, t2j1), (t2j2_h, t2j2),
                     (t2o_h, t2o), (t2c_h, t2c),
                     (t3w_h, t3w), (t3j1_h, t3j1), (t3j2_h, t3j2),
                     (t3j3_h, t3j3), (t3o_h, t3o), (t3c_h, t3c)):
        pltpu.sync_copy(src, dst)

    start = wid * BPW
    pltpu.sync_copy(i0_hbm.at[pl.ds(start * L, BPW * L)], i0_v)

    zero16 = jnp.zeros((L,), jnp.float32)

    def sb_body(sb, _):
        base = jnp.minimum(start + sb * R, NBLK - R)
        rel = base - start
        pltpu.sync_copy(x1_hbm.at[pl.ds(base * BW, R * BW)], x1_v)
        i0x = [i0_v[pl.ds((rel + r) * L, L)] * X0 for r in range(R)]

        @plsc.parallel_loop(0, R * L * X2, L, unroll=8)
        def _z(k):
            out_v[pl.ds(k, L)] = zero16

        @plsc.parallel_loop(0, P1 * L, L, unroll=2)
        def _p1(b):
            wv = t1w[pl.ds(b, L)]
            j1 = t1j1[pl.ds(b, L)]
            ov = t1o[pl.ds(b, L)]
            cv = t1c[pl.ds(b, L)]
            for r in range(R):
                a = plsc.load_gather(x0_v, [i0x[r] + wv])
                v = (cv * a) * plsc.load_gather(x1_v, [j1 + (r * BW)])
                plsc.addupdate_scatter(out_v, [ov + (r * BW)], v)

        @plsc.parallel_loop(0, P2 * L, L, unroll=2)
        def _p2(b):
            wv = t2w[pl.ds(b, L)]
            j1 = t2j1[pl.ds(b, L)]
            j2 = t2j2[pl.ds(b, L)]
            ov = t2o[pl.ds(b, L)]
            cv = t2c[pl.ds(b, L)]
            for r in range(R):
                a = plsc.load_gather(x0_v, [i0x[r] + wv])
                v = ((cv * a) * plsc.load_gather(x1_v, [j1 + (r * BW)])
                     * plsc.load_gather(x1_v, [j2 + (r * BW)]))
                plsc.addupdate_scatter(out_v, [ov + (r * BW)], v)

        @plsc.parallel_loop(0, P3 * L, L, unroll=2)
        def _p3(b):
            wv = t3w[pl.ds(b, L)]
            j1 = t3j1[pl.ds(b, L)]
            j2 = t3j2[pl.ds(b, L)]
            j3 = t3j3[pl.ds(b, L)]
            ov = t3o[pl.ds(b, L)]
            cv = t3c[pl.ds(b, L)]
            for r in range(R):
                a = plsc.load_gather(x0_v, [i0x[r] + wv])
                v = ((cv * a) * plsc.load_gather(x1_v, [j1 + (r * BW)])
                     * (plsc.load_gather(x1_v, [j2 + (r * BW)])
                        * plsc.load_gather(x1_v, [j3 + (r * BW)])))
                plsc.addupdate_scatter(out_v, [ov + (r * BW)], v)

        pltpu.sync_copy(out_v, out_hbm.at[pl.ds(base * BW, R * BW)])
        return 0

    lax.fori_loop(0, NSB, sb_body, 0)


@jax.jit
def kernel(x0, i0, x1, idx1, coeff1, idx2, coeff2, idx3, coeff3):
    n = x1.shape[0]
    f32, i32 = jnp.float32, jnp.int32
    lane = jnp.arange(L, dtype=i32)

    def flat(j):
        return (lane[None, :] * X1 + j[:, None].astype(i32)).reshape(-1)

    def splat(v):
        return jnp.broadcast_to(v[:, None], (v.shape[0], L)).reshape(-1)

    args = [
        x0.reshape(-1), jnp.pad(i0, (0, NPAD - n)), x1.reshape(-1),
        splat(idx1[:, 0]), flat(idx1[:, 1]), flat(idx1[:, 2]), splat(coeff1),
        splat(idx2[:, 0]), flat(idx2[:, 1]), flat(idx2[:, 2]), flat(idx2[:, 3]),
        splat(coeff2),
        splat(idx3[:, 0]), flat(idx3[:, 1]), flat(idx3[:, 2]), flat(idx3[:, 3]),
        flat(idx3[:, 4]), splat(coeff3),
    ]

    run = pl.kernel(
        _sc_body,
        out_type=jax.ShapeDtypeStruct((n * X2,), f32),
        mesh=plsc.VectorSubcoreMesh(core_axis_name="c", subcore_axis_name="s"),
        compiler_params=pltpu.CompilerParams(needs_layout_passes=False),
        scratch_types=[
            pltpu.VMEM((Z * X0,), f32),
            pltpu.VMEM((BPW * L,), i32),
            pltpu.VMEM((R * BW,), f32),
            pltpu.VMEM((R * BW,), f32),
            pltpu.VMEM((P1 * L,), i32), pltpu.VMEM((P1 * L,), i32),
            pltpu.VMEM((P1 * L,), i32), pltpu.VMEM((P1 * L,), f32),
            pltpu.VMEM((P2 * L,), i32), pltpu.VMEM((P2 * L,), i32),
            pltpu.VMEM((P2 * L,), i32), pltpu.VMEM((P2 * L,), i32),
            pltpu.VMEM((P2 * L,), f32),
            pltpu.VMEM((P3 * L,), i32), pltpu.VMEM((P3 * L,), i32),
            pltpu.VMEM((P3 * L,), i32), pltpu.VMEM((P3 * L,), i32),
            pltpu.VMEM((P3 * L,), i32), pltpu.VMEM((P3 * L,), f32),
        ],
    )
    return run(*args).reshape(n, X2)


# speedup vs baseline: 5.4980x; 3.7943x over previous
import functools

import jax
import jax.numpy as jnp
from jax import lax
from jax.experimental import pallas as pl
from jax.experimental.pallas import tpu as pltpu
from jax.experimental.pallas import tpu_sc as plsc

L = 16
NW = 32
Z, X0, X1, X2 = 64, 128, 128, 128
P1, P2, P3 = 64, 128, 256
N = 50000
R = 7
GW = R * L
NG = -(-N // GW)
GPW = -(-NG // NW)
NP = NG * GW
GB = X1 * GW


def _sc_body(x0t_hbm, i0_hbm, x1p_hbm,
             t1w_h, t1j1_h, t1o_h, t1c_h,
             t2w_h, t2j1_h, t2j2_h, t2o_h, t2c_h,
             t3w_h, t3j1_h, t3j2_h, t3j3_h, t3o_h, t3c_h,
             out_hbm,
             x0t_v, x0g_v, i0_v, x1_v, out_v,
             t1w, t1j1, t1o, t1c,
             t2w, t2j1, t2j2, t2o, t2c,
             t3w, t3j1, t3j2, t3j3, t3o, t3c):
    c = lax.axis_index("c")
    s = lax.axis_index("s")
    wid = s * 2 + c

    pltpu.sync_copy(x0t_hbm, x0t_v)
    for src, dst in ((t1w_h, t1w), (t1j1_h, t1j1), (t1o_h, t1o), (t1c_h, t1c),
                     (t2w_h, t2w), (t2j1_h, t2j1), (t2j2_h, t2j2),
                     (t2o_h, t2o), (t2c_h, t2c),
                     (t3w_h, t3w), (t3j1_h, t3j1), (t3j2_h, t3j2),
                     (t3j3_h, t3j3), (t3o_h, t3o), (t3c_h, t3c)):
        pltpu.sync_copy(src, dst)

    g0 = wid * GPW
    pltpu.sync_copy(i0_hbm.at[pl.ds(g0 * GW, GPW * GW)], i0_v)

    zero16 = jnp.zeros((L,), jnp.float32)

    def sb_body(sb, _):
        g = jnp.minimum(g0 + sb, NG - 1)
        rel = g - g0
        pltpu.sync_copy(x1p_hbm.at[pl.ds(g * GB, GB)], x1_v)
        i0v = [i0_v[pl.ds(rel * GW + r * L, L)] for r in range(R)]

        @plsc.parallel_loop(0, X0, 1, unroll=2)
        def _pre(w):
            zbase = w * Z
            obase = w * GW
            for r in range(R):
                row = plsc.load_gather(x0t_v, [zbase + i0v[r]])
                x0g_v[pl.ds(obase + r * L, L)] = row

        @plsc.parallel_loop(0, GB, L, unroll=8)
        def _z(k):
            out_v[pl.ds(k, L)] = zero16

        @plsc.parallel_loop(0, P1 * L, L, unroll=2)
        def _p1(b):
            wv = t1w[pl.ds(b, L)]
            j1 = t1j1[pl.ds(b, L)]
            ov = t1o[pl.ds(b, L)]
            cv = t1c[pl.ds(b, L)]
            for r in range(R):
                a = plsc.load_gather(x0g_v, [wv + (r * L)])
                v = (cv * a) * plsc.load_gather(x1_v, [j1 + (r * L)])
                plsc.addupdate_scatter(out_v, [ov + (r * L)], v)

        @plsc.parallel_loop(0, P2 * L, L, unroll=2)
        def _p2(b):
            wv = t2w[pl.ds(b, L)]
            j1 = t2j1[pl.ds(b, L)]
            j2 = t2j2[pl.ds(b, L)]
            ov = t2o[pl.ds(b, L)]
            cv = t2c[pl.ds(b, L)]
            for r in range(R):
                a = plsc.load_gather(x0g_v, [wv + (r * L)])
                v = ((cv * a) * plsc.load_gather(x1_v, [j1 + (r * L)])
                     * plsc.load_gather(x1_v, [j2 + (r * L)]))
                plsc.addupdate_scatter(out_v, [ov + (r * L)], v)

        @plsc.parallel_loop(0, P3 * L, L, unroll=2)
        def _p3(b):
            wv = t3w[pl.ds(b, L)]
            j1 = t3j1[pl.ds(b, L)]
            j2 = t3j2[pl.ds(b, L)]
            j3 = t3j3[pl.ds(b, L)]
            ov = t3o[pl.ds(b, L)]
            cv = t3c[pl.ds(b, L)]
            for r in range(R):
                a = plsc.load_gather(x0g_v, [wv + (r * L)])
                v = ((cv * a) * plsc.load_gather(x1_v, [j1 + (r * L)])
                     * (plsc.load_gather(x1_v, [j2 + (r * L)])
                        * plsc.load_gather(x1_v, [j3 + (r * L)])))
                plsc.addupdate_scatter(out_v, [ov + (r * L)], v)

        pltpu.sync_copy(out_v, out_hbm.at[pl.ds(g * GB, GB)])
        return 0

    lax.fori_loop(0, GPW, sb_body, 0)


@jax.jit
def kernel(x0, i0, x1, idx1, coeff1, idx2, coeff2, idx3, coeff3):
    n = x1.shape[0]
    f32, i32 = jnp.float32, jnp.int32
    lane = jnp.arange(L, dtype=i32)

    def flat(j):
        return (lane[None, :] + GW * j[:, None].astype(i32)).reshape(-1)

    def splat(v):
        return jnp.broadcast_to(v[:, None], (v.shape[0], L)).reshape(-1)

    x1p = jnp.pad(x1, ((0, NP - n), (0, 0))).reshape(NG, GW, X1)
    x1p = jnp.transpose(x1p, (0, 2, 1)).reshape(-1)

    args = [
        x0.T.reshape(-1), jnp.pad(i0, (0, NP - n)), x1p,
        flat(idx1[:, 0]), flat(idx1[:, 1]), flat(idx1[:, 2]),
        splat(coeff1),
        flat(idx2[:, 0]), flat(idx2[:, 1]), flat(idx2[:, 2]),
        flat(idx2[:, 3]), splat(coeff2),
        flat(idx3[:, 0]), flat(idx3[:, 1]), flat(idx3[:, 2]),
        flat(idx3[:, 3]), flat(idx3[:, 4]), splat(coeff3),
    ]

    run = pl.kernel(
        _sc_body,
        out_type=jax.ShapeDtypeStruct((NG * GB,), f32),
        mesh=plsc.VectorSubcoreMesh(core_axis_name="c", subcore_axis_name="s"),
        compiler_params=pltpu.CompilerParams(needs_layout_passes=False),
        scratch_types=[
            pltpu.VMEM((Z * X0,), f32),
            pltpu.VMEM((X0 * GW,), f32),
            pltpu.VMEM((GPW * GW,), i32),
            pltpu.VMEM((GB,), f32),
            pltpu.VMEM((GB,), f32),
            pltpu.VMEM((P1 * L,), i32), pltpu.VMEM((P1 * L,), i32),
            pltpu.VMEM((P1 * L,), i32), pltpu.VMEM((P1 * L,), f32),
            pltpu.VMEM((P2 * L,), i32), pltpu.VMEM((P2 * L,), i32),
            pltpu.VMEM((P2 * L,), i32), pltpu.VMEM((P2 * L,), i32),
            pltpu.VMEM((P2 * L,), f32),
            pltpu.VMEM((P3 * L,), i32), pltpu.VMEM((P3 * L,), i32),
            pltpu.VMEM((P3 * L,), i32), pltpu.VMEM((P3 * L,), i32),
            pltpu.VMEM((P3 * L,), i32), pltpu.VMEM((P3 * L,), f32),
        ],
    )
    outp = run(*args)
    out = jnp.transpose(outp.reshape(NG, X2, GW), (0, 2, 1)).reshape(NP, X2)
    return out[:n]


# speedup vs baseline: 8.1053x; 1.4742x over previous
import functools

import jax
import jax.numpy as jnp
from jax import lax
from jax.experimental import pallas as pl
from jax.experimental.pallas import tpu as pltpu
from jax.experimental.pallas import tpu_sc as plsc

L = 16
NW = 32
Z, X0, X1, X2 = 64, 128, 128, 128
P1, P2, P3 = 64, 128, 256
N = 50000
TCB = 512
NTB = 70
NT = NTB * TCB
NS = N - NT
R = 7
GW = R * L
NG = -(-NS // GW)
GPW = -(-NG // NW)
NP = NG * GW
GB = X1 * GW


def _sc_body(x0t_hbm, i0_hbm, x1p_hbm,
             t1w_h, t1j1_h, t1o_h, t1c_h,
             t2w_h, t2j1_h, t2j2_h, t2o_h, t2c_h,
             t3w_h, t3j1_h, t3j2_h, t3j3_h, t3o_h, t3c_h,
             out_hbm,
             x0t_v, x0g_v, i0_v, x1_v, out_v,
             t1w, t1j1, t1o, t1c,
             t2w, t2j1, t2j2, t2o, t2c,
             t3w, t3j1, t3j2, t3j3, t3o, t3c):
    c = lax.axis_index("c")
    s = lax.axis_index("s")
    wid = s * 2 + c

    pltpu.sync_copy(x0t_hbm, x0t_v)
    for src, dst in ((t1w_h, t1w), (t1j1_h, t1j1), (t1o_h, t1o), (t1c_h, t1c),
                     (t2w_h, t2w), (t2j1_h, t2j1), (t2j2_h, t2j2),
                     (t2o_h, t2o), (t2c_h, t2c),
                     (t3w_h, t3w), (t3j1_h, t3j1), (t3j2_h, t3j2),
                     (t3j3_h, t3j3), (t3o_h, t3o), (t3c_h, t3c)):
        pltpu.sync_copy(src, dst)

    g0 = wid * GPW
    i0base = jnp.minimum(g0 * GW, NP - GPW * GW)
    pltpu.sync_copy(i0_hbm.at[pl.ds(i0base, GPW * GW)], i0_v)

    zero16 = jnp.zeros((L,), jnp.float32)

    def sb_body(sb, _):
        g = jnp.minimum(g0 + sb, NG - 1)
        rel = g * GW - i0base
        pltpu.sync_copy(x1p_hbm.at[pl.ds(g * GB, GB)], x1_v)
        i0v = [i0_v[pl.ds(rel + r * L, L)] for r in range(R)]

        @plsc.parallel_loop(0, X0, 1, unroll=2)
        def _pre(w):
            zbase = w * Z
            obase = w * GW
            for r in range(R):
                row = plsc.load_gather(x0t_v, [zbase + i0v[r]])
                x0g_v[pl.ds(obase + r * L, L)] = row

        @plsc.parallel_loop(0, GB, L, unroll=8)
        def _z(k):
            out_v[pl.ds(k, L)] = zero16

        @plsc.parallel_loop(0, P1 * L, L, unroll=2)
        def _p1(b):
            wv = t1w[pl.ds(b, L)]
            j1 = t1j1[pl.ds(b, L)]
            ov = t1o[pl.ds(b, L)]
            cv = t1c[pl.ds(b, L)]
            for r in range(R):
                a = plsc.load_gather(x0g_v, [wv + (r * L)])
                v = (cv * a) * plsc.load_gather(x1_v, [j1 + (r * L)])
                plsc.addupdate_scatter(out_v, [ov + (r * L)], v)

        @plsc.parallel_loop(0, P2 * L, L, unroll=2)
        def _p2(b):
            wv = t2w[pl.ds(b, L)]
            j1 = t2j1[pl.ds(b, L)]
            j2 = t2j2[pl.ds(b, L)]
            ov = t2o[pl.ds(b, L)]
            cv = t2c[pl.ds(b, L)]
            for r in range(R):
                a = plsc.load_gather(x0g_v, [wv + (r * L)])
                v = ((cv * a) * plsc.load_gather(x1_v, [j1 + (r * L)])
                     * plsc.load_gather(x1_v, [j2 + (r * L)]))
                plsc.addupdate_scatter(out_v, [ov + (r * L)], v)

        @plsc.parallel_loop(0, P3 * L, L, unroll=2)
        def _p3(b):
            wv = t3w[pl.ds(b, L)]
            j1 = t3j1[pl.ds(b, L)]
            j2 = t3j2[pl.ds(b, L)]
            j3 = t3j3[pl.ds(b, L)]
            ov = t3o[pl.ds(b, L)]
            cv = t3c[pl.ds(b, L)]
            for r in range(R):
                a = plsc.load_gather(x0g_v, [wv + (r * L)])
                v = ((cv * a) * plsc.load_gather(x1_v, [j1 + (r * L)])
                     * (plsc.load_gather(x1_v, [j2 + (r * L)])
                        * plsc.load_gather(x1_v, [j3 + (r * L)])))
                plsc.addupdate_scatter(out_v, [ov + (r * L)], v)

        pltpu.sync_copy(out_v, out_hbm.at[pl.ds(g * GB, GB)])
        return 0

    lax.fori_loop(0, GPW, sb_body, 0)


def _sc_part(x0, i0_tail, x1_tail, idx1, coeff1, idx2, coeff2, idx3, coeff3):
    f32, i32 = jnp.float32, jnp.int32
    lane = jnp.arange(L, dtype=i32)

    def flat(j):
        return (lane[None, :] + GW * j[:, None].astype(i32)).reshape(-1)

    def splat(v):
        return jnp.broadcast_to(v[:, None], (v.shape[0], L)).reshape(-1)

    x1p = jnp.pad(x1_tail, ((0, NP - NS), (0, 0))).reshape(NG, GW, X1)
    x1p = jnp.transpose(x1p, (0, 2, 1)).reshape(-1)

    args = [
        x0.T.reshape(-1), jnp.pad(i0_tail, (0, NP - NS)), x1p,
        flat(idx1[:, 0]), flat(idx1[:, 1]), flat(idx1[:, 2]),
        splat(coeff1),
        flat(idx2[:, 0]), flat(idx2[:, 1]), flat(idx2[:, 2]),
        flat(idx2[:, 3]), splat(coeff2),
        flat(idx3[:, 0]), flat(idx3[:, 1]), flat(idx3[:, 2]),
        flat(idx3[:, 3]), flat(idx3[:, 4]), splat(coeff3),
    ]

    run = pl.kernel(
        _sc_body,
        out_type=jax.ShapeDtypeStruct((NG * GB,), f32),
        mesh=plsc.VectorSubcoreMesh(core_axis_name="c", subcore_axis_name="s"),
        compiler_params=pltpu.CompilerParams(needs_layout_passes=False),
        scratch_types=[
            pltpu.VMEM((Z * X0,), f32),
            pltpu.VMEM((X0 * GW,), f32),
            pltpu.VMEM((GPW * GW,), i32),
            pltpu.VMEM((GB,), f32),
            pltpu.VMEM((GB,), f32),
            pltpu.VMEM((P1 * L,), i32), pltpu.VMEM((P1 * L,), i32),
            pltpu.VMEM((P1 * L,), i32), pltpu.VMEM((P1 * L,), f32),
            pltpu.VMEM((P2 * L,), i32), pltpu.VMEM((P2 * L,), i32),
            pltpu.VMEM((P2 * L,), i32), pltpu.VMEM((P2 * L,), i32),
            pltpu.VMEM((P2 * L,), f32),
            pltpu.VMEM((P3 * L,), i32), pltpu.VMEM((P3 * L,), i32),
            pltpu.VMEM((P3 * L,), i32), pltpu.VMEM((P3 * L,), i32),
            pltpu.VMEM((P3 * L,), i32), pltpu.VMEM((P3 * L,), f32),
        ],
    )
    outp = run(*args)
    out = jnp.transpose(outp.reshape(NG, X2, GW), (0, 2, 1)).reshape(NP, X2)
    return out[:NS]


def _onehot_cols(idx, depth, dtype=jnp.float32):
    return (idx[None, :] == jnp.arange(depth, dtype=idx.dtype)[:, None]).astype(dtype)


def _tc_body(x0_ref, i0_ref, x1_ref,
             gw1_ref, gj1_ref, s1_ref,
             gw2_ref, ga2_ref, gb2_ref, s2_ref,
             gw3_ref, ga3_ref, gb3_ref, gc3_ref, s3_ref,
             out_ref):
    f32 = jnp.float32
    x1 = x1_ref[...]
    i0 = i0_ref[0, 0, :]
    oh = (i0[:, None] == jax.lax.broadcasted_iota(jnp.int32, (1, Z), 1)).astype(f32)
    x0g = jnp.dot(oh, x0_ref[...], preferred_element_type=f32)

    def deg(gw, *gjs_and_s):
        gjs, sm = gjs_and_s[:-1], gjs_and_s[-1]
        v = jnp.dot(x0g, gw, preferred_element_type=f32)
        for gj in gjs:
            v = v * jnp.dot(x1, gj, preferred_element_type=f32)
        return jnp.dot(v, sm, preferred_element_type=f32)

    acc = deg(gw1_ref[...], gj1_ref[...], s1_ref[...])
    acc += deg(gw2_ref[...], ga2_ref[...], gb2_ref[...], s2_ref[...])
    acc += deg(gw3_ref[...], ga3_ref[...], gb3_ref[...], gc3_ref[...], s3_ref[...])
    out_ref[...] = acc


def _tc_part(x0, i0, x1, idx1, coeff1, idx2, coeff2, idx3, coeff3):
    gw1 = _onehot_cols(idx1[:, 0], X0)
    gj1 = _onehot_cols(idx1[:, 1], X1)
    s1 = _onehot_cols(idx1[:, 2], X2).T * coeff1[:, None]
    gw2 = _onehot_cols(idx2[:, 0], X0)
    ga2 = _onehot_cols(idx2[:, 1], X1)
    gb2 = _onehot_cols(idx2[:, 2], X1)
    s2 = _onehot_cols(idx2[:, 3], X2).T * coeff2[:, None]
    gw3 = _onehot_cols(idx3[:, 0], X0)
    ga3 = _onehot_cols(idx3[:, 1], X1)
    gb3 = _onehot_cols(idx3[:, 2], X1)
    gc3 = _onehot_cols(idx3[:, 3], X1)
    s3 = _onehot_cols(idx3[:, 4], X2).T * coeff3[:, None]

    i0_3d = i0[:NT].reshape(NTB, 1, TCB)

    full = lambda shape: pl.BlockSpec(shape, lambda b: (0,) * len(shape))
    return pl.pallas_call(
        _tc_body,
        grid=(NTB,),
        in_specs=[
            full((Z, X0)),
            pl.BlockSpec((1, 1, TCB), lambda b: (b, 0, 0)),
            pl.BlockSpec((TCB, X1), lambda b: (b, 0)),
            full((X0, 64)), full((X1, 64)), full((64, X2)),
            full((X0, 128)), full((X1, 128)), full((X1, 128)), full((128, X2)),
            full((X0, 256)), full((X1, 256)), full((X1, 256)), full((X1, 256)),
            full((256, X2)),
        ],
        out_specs=pl.BlockSpec((TCB, X2), lambda b: (b, 0)),
        out_shape=jax.ShapeDtypeStruct((NT, X2), x1.dtype),
    )(x0, i0_3d, x1[:NT],
      gw1, gj1, s1, gw2, ga2, gb2, s2, gw3, ga3, gb3, gc3, s3)


@jax.jit
def kernel(x0, i0, x1, idx1, coeff1, idx2, coeff2, idx3, coeff3):
    out_sc = _sc_part(x0, i0[NT:], x1[NT:], idx1, coeff1, idx2, coeff2,
                      idx3, coeff3)
    out_tc = _tc_part(x0, i0, x1, idx1, coeff1, idx2, coeff2, idx3, coeff3)
    return jnp.concatenate([out_tc, out_sc], axis=0)


# speedup vs baseline: 8.6419x; 1.0662x over previous
import functools

import jax
import jax.numpy as jnp
from jax import lax
from jax.experimental import pallas as pl
from jax.experimental.pallas import tpu as pltpu
from jax.experimental.pallas import tpu_sc as plsc

L = 16
NW = 32
Z, X0, X1, X2 = 64, 128, 128, 128
P1, P2, P3 = 64, 128, 256
N = 50000
TCB = 512
NTB = 70
NT = NTB * TCB
NS = N - NT
R = 7
GW = R * L
NG = -(-NS // GW)
GPW = -(-NG // NW)
NP = NG * GW
GB = X1 * GW


def _sc_body(x0t_hbm, i0_hbm, x1p_hbm,
             t1w_h, t1j1_h, t1o_h, t1c_h,
             t2w_h, t2j1_h, t2j2_h, t2o_h, t2c_h,
             t3w_h, t3j1_h, t3j2_h, t3j3_h, t3o_h, t3c_h,
             out_hbm,
             x0t_v, x0g_v, i0_v, x1_v, out_v,
             t1w, t1j1, t1o, t1c,
             t2w, t2j1, t2j2, t2o, t2c,
             t3w, t3j1, t3j2, t3j3, t3o, t3c):
    c = lax.axis_index("c")
    s = lax.axis_index("s")
    wid = s * 2 + c

    pltpu.sync_copy(x0t_hbm, x0t_v)
    for src, dst in ((t1w_h, t1w), (t1j1_h, t1j1), (t1o_h, t1o), (t1c_h, t1c),
                     (t2w_h, t2w), (t2j1_h, t2j1), (t2j2_h, t2j2),
                     (t2o_h, t2o), (t2c_h, t2c),
                     (t3w_h, t3w), (t3j1_h, t3j1), (t3j2_h, t3j2),
                     (t3j3_h, t3j3), (t3o_h, t3o), (t3c_h, t3c)):
        pltpu.sync_copy(src, dst)

    g0 = wid * GPW
    i0base = jnp.minimum(g0 * GW, NP - GPW * GW)
    pltpu.sync_copy(i0_hbm.at[pl.ds(i0base, GPW * GW)], i0_v)

    zero16 = jnp.zeros((L,), jnp.float32)

    def sb_body(sb, _):
        g = jnp.minimum(g0 + sb, NG - 1)
        rel = g * GW - i0base
        pltpu.sync_copy(x1p_hbm.at[pl.ds(g * GB, GB)], x1_v)
        i0v = [i0_v[pl.ds(rel + r * L, L)] for r in range(R)]

        @plsc.parallel_loop(0, X0, 1, unroll=2)
        def _pre(w):
            zbase = w * Z
            obase = w * GW
            for r in range(R):
                row = plsc.load_gather(x0t_v, [zbase + i0v[r]])
                x0g_v[pl.ds(obase + r * L, L)] = row

        @plsc.parallel_loop(0, GB, L, unroll=8)
        def _z(k):
            out_v[pl.ds(k, L)] = zero16

        @plsc.parallel_loop(0, P1 * L, L, unroll=2)
        def _p1(b):
            wv = t1w[pl.ds(b, L)]
            j1 = t1j1[pl.ds(b, L)]
            ov = t1o[pl.ds(b, L)]
            cv = t1c[pl.ds(b, L)]
            for r in range(R):
                a = plsc.load_gather(x0g_v, [wv + (r * L)])
                v = (cv * a) * plsc.load_gather(x1_v, [j1 + (r * L)])
                plsc.addupdate_scatter(out_v, [ov + (r * L)], v)

        @plsc.parallel_loop(0, P2 * L, L, unroll=2)
        def _p2(b):
            wv = t2w[pl.ds(b, L)]
            j1 = t2j1[pl.ds(b, L)]
            j2 = t2j2[pl.ds(b, L)]
            ov = t2o[pl.ds(b, L)]
            cv = t2c[pl.ds(b, L)]
            for r in range(R):
                a = plsc.load_gather(x0g_v, [wv + (r * L)])
                v = ((cv * a) * plsc.load_gather(x1_v, [j1 + (r * L)])
                     * plsc.load_gather(x1_v, [j2 + (r * L)]))
                plsc.addupdate_scatter(out_v, [ov + (r * L)], v)

        @plsc.parallel_loop(0, P3 * L, L, unroll=2)
        def _p3(b):
            wv = t3w[pl.ds(b, L)]
            j1 = t3j1[pl.ds(b, L)]
            j2 = t3j2[pl.ds(b, L)]
            j3 = t3j3[pl.ds(b, L)]
            ov = t3o[pl.ds(b, L)]
            cv = t3c[pl.ds(b, L)]
            for r in range(R):
                a = plsc.load_gather(x0g_v, [wv + (r * L)])
                v = ((cv * a) * plsc.load_gather(x1_v, [j1 + (r * L)])
                     * (plsc.load_gather(x1_v, [j2 + (r * L)])
                        * plsc.load_gather(x1_v, [j3 + (r * L)])))
                plsc.addupdate_scatter(out_v, [ov + (r * L)], v)

        pltpu.sync_copy(out_v, out_hbm.at[pl.ds(g * GB, GB)])
        return 0

    lax.fori_loop(0, GPW, sb_body, 0)


def _sc_part(x0, i0_tail, x1_tail, idx1, coeff1, idx2, coeff2, idx3, coeff3):
    f32, i32 = jnp.float32, jnp.int32
    lane = jnp.arange(L, dtype=i32)

    def flat(j):
        return (lane[None, :] + GW * j[:, None].astype(i32)).reshape(-1)

    def splat(v):
        return jnp.broadcast_to(v[:, None], (v.shape[0], L)).reshape(-1)

    x1p = jnp.pad(x1_tail, ((0, NP - NS), (0, 0))).reshape(NG, GW, X1)
    x1p = jnp.transpose(x1p, (0, 2, 1)).reshape(-1)

    args = [
        x0.T.reshape(-1), jnp.pad(i0_tail, (0, NP - NS)), x1p,
        flat(idx1[:, 0]), flat(idx1[:, 1]), flat(idx1[:, 2]),
        splat(coeff1),
        flat(idx2[:, 0]), flat(idx2[:, 1]), flat(idx2[:, 2]),
        flat(idx2[:, 3]), splat(coeff2),
        flat(idx3[:, 0]), flat(idx3[:, 1]), flat(idx3[:, 2]),
        flat(idx3[:, 3]), flat(idx3[:, 4]), splat(coeff3),
    ]

    run = pl.kernel(
        _sc_body,
        out_type=jax.ShapeDtypeStruct((NG * GB,), f32),
        mesh=plsc.VectorSubcoreMesh(core_axis_name="c", subcore_axis_name="s"),
        compiler_params=pltpu.CompilerParams(needs_layout_passes=False),
        scratch_types=[
            pltpu.VMEM((Z * X0,), f32),
            pltpu.VMEM((X0 * GW,), f32),
            pltpu.VMEM((GPW * GW,), i32),
            pltpu.VMEM((GB,), f32),
            pltpu.VMEM((GB,), f32),
            pltpu.VMEM((P1 * L,), i32), pltpu.VMEM((P1 * L,), i32),
            pltpu.VMEM((P1 * L,), i32), pltpu.VMEM((P1 * L,), f32),
            pltpu.VMEM((P2 * L,), i32), pltpu.VMEM((P2 * L,), i32),
            pltpu.VMEM((P2 * L,), i32), pltpu.VMEM((P2 * L,), i32),
            pltpu.VMEM((P2 * L,), f32),
            pltpu.VMEM((P3 * L,), i32), pltpu.VMEM((P3 * L,), i32),
            pltpu.VMEM((P3 * L,), i32), pltpu.VMEM((P3 * L,), i32),
            pltpu.VMEM((P3 * L,), i32), pltpu.VMEM((P3 * L,), f32),
        ],
    )
    outp = run(*args)
    out = jnp.transpose(outp.reshape(NG, X2, GW), (0, 2, 1)).reshape(NP, X2)
    return out[:NS]


def _onehot_cols(idx, depth, dtype=jnp.float32):
    return (idx[None, :] == jnp.arange(depth, dtype=idx.dtype)[:, None]).astype(dtype)


def _tc_body(x0_ref, i0_ref, x1_ref,
             gw1_ref, gj1_ref, s1_ref,
             gw2_ref, ga2_ref, gb2_ref, s2_ref,
             gw3_ref, ga3_ref, gb3_ref, gc3_ref, s3_ref,
             out_ref):
    f32 = jnp.float32
    x1 = x1_ref[...]
    i0 = i0_ref[0, 0, :]
    oh = (i0[:, None] == jax.lax.broadcasted_iota(jnp.int32, (1, Z), 1)).astype(f32)
    x0g = jnp.dot(oh, x0_ref[...], preferred_element_type=f32)

    def deg(gw, *gjs_and_s):
        gjs, sm = gjs_and_s[:-1], gjs_and_s[-1]
        v = jnp.dot(x0g, gw, preferred_element_type=f32)
        for gj in gjs:
            v = v * jnp.dot(x1, gj, preferred_element_type=f32)
        return jnp.dot(v, sm, preferred_element_type=f32)

    acc = deg(gw1_ref[...], gj1_ref[...], s1_ref[...])
    acc += deg(gw2_ref[...], ga2_ref[...], gb2_ref[...], s2_ref[...])
    acc += deg(gw3_ref[...], ga3_ref[...], gb3_ref[...], gc3_ref[...], s3_ref[...])
    out_ref[...] = acc


def _tc_part(x0, i0, x1, idx1, coeff1, idx2, coeff2, idx3, coeff3):
    gw1 = _onehot_cols(idx1[:, 0], X0)
    gj1 = _onehot_cols(idx1[:, 1], X1)
    s1 = _onehot_cols(idx1[:, 2], X2).T * coeff1[:, None]
    gw2 = _onehot_cols(idx2[:, 0], X0)
    ga2 = _onehot_cols(idx2[:, 1], X1)
    gb2 = _onehot_cols(idx2[:, 2], X1)
    s2 = _onehot_cols(idx2[:, 3], X2).T * coeff2[:, None]
    gw3 = _onehot_cols(idx3[:, 0], X0)
    ga3 = _onehot_cols(idx3[:, 1], X1)
    gb3 = _onehot_cols(idx3[:, 2], X1)
    gc3 = _onehot_cols(idx3[:, 3], X1)
    s3 = _onehot_cols(idx3[:, 4], X2).T * coeff3[:, None]

    i0_3d = i0[:NT].reshape(NTB, 1, TCB)

    full = lambda shape: pl.BlockSpec(shape, lambda b: (0,) * len(shape))
    return pl.pallas_call(
        _tc_body,
        grid=(NTB,),
        in_specs=[
            full((Z, X0)),
            pl.BlockSpec((1, 1, TCB), lambda b: (b, 0, 0)),
            pl.BlockSpec((TCB, X1), lambda b: (b, 0)),
            full((X0, 64)), full((X1, 64)), full((64, X2)),
            full((X0, 128)), full((X1, 128)), full((X1, 128)), full((128, X2)),
            full((X0, 256)), full((X1, 256)), full((X1, 256)), full((X1, 256)),
            full((256, X2)),
        ],
        out_specs=pl.BlockSpec((TCB, X2), lambda b: (b, 0)),
        out_shape=jax.ShapeDtypeStruct((NT, X2), x1.dtype),
    )(x0, i0_3d, x1,
      gw1, gj1, s1, gw2, ga2, gb2, s2, gw3, ga3, gb3, gc3, s3)


@jax.jit
def kernel(x0, i0, x1, idx1, coeff1, idx2, coeff2, idx3, coeff3):
    out_tc = _tc_part(x0, i0, x1, idx1, coeff1, idx2, coeff2, idx3, coeff3)
    out_sc = _sc_part(x0, i0[NT:], x1[NT:], idx1, coeff1, idx2, coeff2,
                      idx3, coeff3)
    return jnp.concatenate([out_tc, out_sc], axis=0)


# speedup vs baseline: 8.6573x; 1.0018x over previous
import functools

import jax
import jax.numpy as jnp
from jax import lax
from jax.experimental import pallas as pl
from jax.experimental.pallas import tpu as pltpu
from jax.experimental.pallas import tpu_sc as plsc

L = 16
NW = 32
Z, X0, X1, X2 = 64, 128, 128, 128
P1, P2, P3 = 64, 128, 256
N = 50000
TCB = 512
NTB = 70
NT = NTB * TCB
NS = N - NT
R = 7
GW = R * L
NG = -(-NS // GW)
GPW = -(-NG // NW)
NP = NG * GW
GB = X1 * GW


def _sc_body(x0t_hbm, i0_hbm, x1p_hbm,
             t1w_h, t1j1_h, t1o_h, t1c_h,
             t2w_h, t2j1_h, t2j2_h, t2o_h, t2c_h,
             t3w_h, t3j1_h, t3j2_h, t3j3_h, t3o_h, t3c_h,
             out_hbm,
             x0t_v, x0g_v, i0_v, x1_v, out_v,
             t1w, t1j1, t1o, t1c,
             t2w, t2j1, t2j2, t2o, t2c,
             t3w, t3j1, t3j2, t3j3, t3o, t3c):
    c = lax.axis_index("c")
    s = lax.axis_index("s")
    wid = s * 2 + c

    pltpu.sync_copy(x0t_hbm, x0t_v)
    for src, dst in ((t1w_h, t1w), (t1j1_h, t1j1), (t1o_h, t1o), (t1c_h, t1c),
                     (t2w_h, t2w), (t2j1_h, t2j1), (t2j2_h, t2j2),
                     (t2o_h, t2o), (t2c_h, t2c),
                     (t3w_h, t3w), (t3j1_h, t3j1), (t3j2_h, t3j2),
                     (t3j3_h, t3j3), (t3o_h, t3o), (t3c_h, t3c)):
        pltpu.sync_copy(src, dst)

    g0 = wid * GPW
    i0base = jnp.minimum(g0 * GW, NP - GPW * GW)
    pltpu.sync_copy(i0_hbm.at[pl.ds(i0base, GPW * GW)], i0_v)

    zero16 = jnp.zeros((L,), jnp.float32)

    def sb_body(sb, _):
        g = jnp.minimum(g0 + sb, NG - 1)
        rel = g * GW - i0base
        pltpu.sync_copy(x1p_hbm.at[pl.ds(g * GB, GB)], x1_v)
        i0v = [i0_v[pl.ds(rel + r * L, L)] for r in range(R)]

        @plsc.parallel_loop(0, X0, 1, unroll=2)
        def _pre(w):
            zbase = w * Z
            obase = w * GW
            for r in range(R):
                row = plsc.load_gather(x0t_v, [zbase + i0v[r]])
                x0g_v[pl.ds(obase + r * L, L)] = row

        @plsc.parallel_loop(0, GB, L, unroll=8)
        def _z(k):
            out_v[pl.ds(k, L)] = zero16

        @plsc.parallel_loop(0, P1 * L, L, unroll=2)
        def _p1(b):
            wv = t1w[pl.ds(b, L)]
            j1 = t1j1[pl.ds(b, L)]
            ov = t1o[pl.ds(b, L)]
            cv = t1c[pl.ds(b, L)]
            for r in range(R):
                a = plsc.load_gather(x0g_v, [wv + (r * L)])
                v = (cv * a) * plsc.load_gather(x1_v, [j1 + (r * L)])
                plsc.addupdate_scatter(out_v, [ov + (r * L)], v)

        @plsc.parallel_loop(0, P2 * L, L, unroll=2)
        def _p2(b):
            wv = t2w[pl.ds(b, L)]
            j1 = t2j1[pl.ds(b, L)]
            j2 = t2j2[pl.ds(b, L)]
            ov = t2o[pl.ds(b, L)]
            cv = t2c[pl.ds(b, L)]
            for r in range(R):
                a = plsc.load_gather(x0g_v, [wv + (r * L)])
                v = ((cv * a) * plsc.load_gather(x1_v, [j1 + (r * L)])
                     * plsc.load_gather(x1_v, [j2 + (r * L)]))
                plsc.addupdate_scatter(out_v, [ov + (r * L)], v)

        @plsc.parallel_loop(0, P3 * L, L, unroll=2)
        def _p3(b):
            wv = t3w[pl.ds(b, L)]
            j1 = t3j1[pl.ds(b, L)]
            j2 = t3j2[pl.ds(b, L)]
            j3 = t3j3[pl.ds(b, L)]
            ov = t3o[pl.ds(b, L)]
            cv = t3c[pl.ds(b, L)]
            for r in range(R):
                a = plsc.load_gather(x0g_v, [wv + (r * L)])
                v = ((cv * a) * plsc.load_gather(x1_v, [j1 + (r * L)])
                     * (plsc.load_gather(x1_v, [j2 + (r * L)])
                        * plsc.load_gather(x1_v, [j3 + (r * L)])))
                plsc.addupdate_scatter(out_v, [ov + (r * L)], v)

        pltpu.sync_copy(out_v, out_hbm.at[pl.ds(g * GB, GB)])
        return 0

    lax.fori_loop(0, GPW, sb_body, 0)


def _sc_part(x0, i0_tail, x1_tail, idx1, coeff1, idx2, coeff2, idx3, coeff3):
    f32, i32 = jnp.float32, jnp.int32
    lane = jnp.arange(L, dtype=i32)

    def flat(j):
        return (lane[None, :] + GW * j[:, None].astype(i32)).reshape(-1)

    def splat(v):
        return jnp.broadcast_to(v[:, None], (v.shape[0], L)).reshape(-1)

    x1p = jnp.pad(x1_tail, ((0, NP - NS), (0, 0))).reshape(NG, GW, X1)
    x1p = jnp.transpose(x1p, (0, 2, 1)).reshape(-1)

    args = [
        x0.T.reshape(-1), jnp.pad(i0_tail, (0, NP - NS)), x1p,
        flat(idx1[:, 0]), flat(idx1[:, 1]), flat(idx1[:, 2]),
        splat(coeff1),
        flat(idx2[:, 0]), flat(idx2[:, 1]), flat(idx2[:, 2]),
        flat(idx2[:, 3]), splat(coeff2),
        flat(idx3[:, 0]), flat(idx3[:, 1]), flat(idx3[:, 2]),
        flat(idx3[:, 3]), flat(idx3[:, 4]), splat(coeff3),
    ]

    run = pl.kernel(
        _sc_body,
        out_type=jax.ShapeDtypeStruct((NG * GB,), f32),
        mesh=plsc.VectorSubcoreMesh(core_axis_name="c", subcore_axis_name="s"),
        compiler_params=pltpu.CompilerParams(needs_layout_passes=False,
                                             skip_device_barrier=True),
        scratch_types=[
            pltpu.VMEM((Z * X0,), f32),
            pltpu.VMEM((X0 * GW,), f32),
            pltpu.VMEM((GPW * GW,), i32),
            pltpu.VMEM((GB,), f32),
            pltpu.VMEM((GB,), f32),
            pltpu.VMEM((P1 * L,), i32), pltpu.VMEM((P1 * L,), i32),
            pltpu.VMEM((P1 * L,), i32), pltpu.VMEM((P1 * L,), f32),
            pltpu.VMEM((P2 * L,), i32), pltpu.VMEM((P2 * L,), i32),
            pltpu.VMEM((P2 * L,), i32), pltpu.VMEM((P2 * L,), i32),
            pltpu.VMEM((P2 * L,), f32),
            pltpu.VMEM((P3 * L,), i32), pltpu.VMEM((P3 * L,), i32),
            pltpu.VMEM((P3 * L,), i32), pltpu.VMEM((P3 * L,), i32),
            pltpu.VMEM((P3 * L,), i32), pltpu.VMEM((P3 * L,), f32),
        ],
    )
    outp = run(*args)
    out = jnp.transpose(outp.reshape(NG, X2, GW), (0, 2, 1)).reshape(NP, X2)
    return out[:NS]


def _onehot_cols(idx, depth, dtype=jnp.float32):
    return (idx[None, :] == jnp.arange(depth, dtype=idx.dtype)[:, None]).astype(dtype)


def _tc_body(x0_ref, i0_ref, x1_ref,
             gw1_ref, gj1_ref, s1_ref,
             gw2_ref, ga2_ref, gb2_ref, s2_ref,
             gw3_ref, ga3_ref, gb3_ref, gc3_ref, s3_ref,
             out_ref):
    f32 = jnp.float32
    x1 = x1_ref[...]
    i0 = i0_ref[0, 0, :]
    oh = (i0[:, None] == jax.lax.broadcasted_iota(jnp.int32, (1, Z), 1)).astype(f32)
    x0g = jnp.dot(oh, x0_ref[...], preferred_element_type=f32)

    def deg(gw, *gjs_and_s):
        gjs, sm = gjs_and_s[:-1], gjs_and_s[-1]
        v = jnp.dot(x0g, gw, preferred_element_type=f32)
        for gj in gjs:
            v = v * jnp.dot(x1, gj, preferred_element_type=f32)
        return jnp.dot(v, sm, preferred_element_type=f32)

    acc = deg(gw1_ref[...], gj1_ref[...], s1_ref[...])
    acc += deg(gw2_ref[...], ga2_ref[...], gb2_ref[...], s2_ref[...])
    acc += deg(gw3_ref[...], ga3_ref[...], gb3_ref[...], gc3_ref[...], s3_ref[...])
    out_ref[...] = acc


def _tc_part(x0, i0, x1, idx1, coeff1, idx2, coeff2, idx3, coeff3):
    gw1 = _onehot_cols(idx1[:, 0], X0)
    gj1 = _onehot_cols(idx1[:, 1], X1)
    s1 = _onehot_cols(idx1[:, 2], X2).T * coeff1[:, None]
    gw2 = _onehot_cols(idx2[:, 0], X0)
    ga2 = _onehot_cols(idx2[:, 1], X1)
    gb2 = _onehot_cols(idx2[:, 2], X1)
    s2 = _onehot_cols(idx2[:, 3], X2).T * coeff2[:, None]
    gw3 = _onehot_cols(idx3[:, 0], X0)
    ga3 = _onehot_cols(idx3[:, 1], X1)
    gb3 = _onehot_cols(idx3[:, 2], X1)
    gc3 = _onehot_cols(idx3[:, 3], X1)
    s3 = _onehot_cols(idx3[:, 4], X2).T * coeff3[:, None]

    i0_3d = i0[:NT].reshape(NTB, 1, TCB)

    full = lambda shape: pl.BlockSpec(shape, lambda b: (0,) * len(shape))
    return pl.pallas_call(
        _tc_body,
        grid=(NTB,),
        in_specs=[
            full((Z, X0)),
            pl.BlockSpec((1, 1, TCB), lambda b: (b, 0, 0)),
            pl.BlockSpec((TCB, X1), lambda b: (b, 0)),
            full((X0, 64)), full((X1, 64)), full((64, X2)),
            full((X0, 128)), full((X1, 128)), full((X1, 128)), full((128, X2)),
            full((X0, 256)), full((X1, 256)), full((X1, 256)), full((X1, 256)),
            full((256, X2)),
        ],
        out_specs=pl.BlockSpec((TCB, X2), lambda b: (b, 0)),
        out_shape=jax.ShapeDtypeStruct((NT, X2), x1.dtype),
        compiler_params=pltpu.CompilerParams(skip_device_barrier=True),
    )(x0, i0_3d, x1,
      gw1, gj1, s1, gw2, ga2, gb2, s2, gw3, ga3, gb3, gc3, s3)


@jax.jit
def kernel(x0, i0, x1, idx1, coeff1, idx2, coeff2, idx3, coeff3):
    out_tc = _tc_part(x0, i0, x1, idx1, coeff1, idx2, coeff2, idx3, coeff3)
    out_sc = _sc_part(x0, i0[NT:], x1[NT:], idx1, coeff1, idx2, coeff2,
                      idx3, coeff3)
    return jnp.concatenate([out_tc, out_sc], axis=0)
